# Initial kernel scaffold; baseline (speedup 1.0000x reference)
import jax
import jax.numpy as jnp
from jax.experimental import pallas as pl


def kernel(features, edge_weights, threashold, conv1_Wl, conv1_bl, conv1_Wr, conv1_br, conv1_We, conv1_att, conv1_bias, conv2_Wl, conv2_bl, conv2_Wr, conv2_br, conv2_We, conv2_att, conv2_bias, conv3_Wl, conv3_bl, conv3_Wr, conv3_br, conv3_We, conv3_att, conv3_bias, conv4_Wl, conv4_bl, conv4_Wr, conv4_br, conv4_We, conv4_att, conv4_bias, c1_w, c1_b, c2_w, c2_b, c3_w, c3_b, l1_W, l1_b):
    raise NotImplementedError("write your pallas kernel here")


# speedup vs baseline: 241.4616x; 241.4616x over previous
import functools

import jax
import jax.numpy as jnp
from jax.experimental import pallas as pl
from jax.experimental.pallas import tpu as pltpu

_N = 1024


def _gat_layer_kernel(cut_ref, x_ref, xT_ref, ewT_ref, Wl_ref, bl_ref, Wr_ref,
                      br_ref, WeT_ref, att_ref, bias_ref, o_ref, xlT_s):
    cut = cut_ref[0, 0]
    ewT_b = ewT_ref[...]
    xT = xT_ref[...]
    x_b = x_ref[...]
    att = att_ref[...]
    WeT = WeT_ref[...]

    xlT = jnp.dot(Wl_ref[...], xT, preferred_element_type=jnp.float32) + bl_ref[...]
    xlT_s[...] = xlT
    xr_b = jax.lax.dot_general(x_b, Wr_ref[...], (((1,), (1,)), ((), ())),
                               preferred_element_type=jnp.float32) + br_ref[...]

    A_row = jnp.dot(att, xlT, preferred_element_type=jnp.float32)
    B_col = jax.lax.dot_general(xr_b, att, (((1,), (1,)), ((), ())),
                                preferred_element_type=jnp.float32)
    S = jax.lax.dot_general(att, WeT, (((1,), (1,)), ((), ())),
                            preferred_element_type=jnp.float32)
    lin = ewT_b * S + A_row + B_col

    d = att.shape[1]

    def body(c, acc):
        sel = (jax.lax.broadcasted_iota(jnp.int32, (1, d), 1) == c).astype(jnp.float32)
        colc = jnp.sum(xr_b * sel, axis=1, keepdims=True)
        wec = jnp.sum(WeT * sel, axis=1, keepdims=True)
        attc = jnp.sum(att * sel, axis=1, keepdims=True)
        rowc = xlT_s[pl.ds(c, 1), :]
        m = ewT_b * wec + rowc + colc
        return acc + jnp.abs(m) * attc

    T = jax.lax.fori_loop(0, d, body, jnp.zeros_like(ewT_b))
    logits = 0.6 * lin + 0.4 * T

    neginf = jnp.float32(-jnp.inf)
    logits = jnp.where(ewT_b > cut, logits, neginf)
    rmax = jnp.max(logits, axis=1, keepdims=True)
    cm = jnp.where(rmax > neginf, rmax, jnp.float32(0.0))
    ex = jnp.exp(logits - cm)
    den = jnp.sum(ex, axis=1, keepdims=True)
    alpha = ex / (den + jnp.float32(1e-16))
    out = jax.lax.dot_general(alpha, xlT, (((1,), (1,)), ((), ())),
                              preferred_element_type=jnp.float32)
    o_ref[...] = jnp.maximum(out + bias_ref[...], jnp.float32(0.0))


def _gat_layer(x, xT, ewT, cutoff, Wl, bl, Wr, br, We, att, bias, *, bj=256):
    n, cin = x.shape
    d = Wl.shape[0]
    grid = (n // bj,)
    return pl.pallas_call(
        _gat_layer_kernel,
        grid=grid,
        in_specs=[
            pl.BlockSpec(memory_space=pltpu.SMEM),
            pl.BlockSpec((bj, cin), lambda j: (j, 0)),
            pl.BlockSpec((cin, n), lambda j: (0, 0)),
            pl.BlockSpec((bj, n), lambda j: (j, 0)),
            pl.BlockSpec((d, cin), lambda j: (0, 0)),
            pl.BlockSpec((d, 1), lambda j: (0, 0)),
            pl.BlockSpec((d, cin), lambda j: (0, 0)),
            pl.BlockSpec((1, d), lambda j: (0, 0)),
            pl.BlockSpec((1, d), lambda j: (0, 0)),
            pl.BlockSpec((1, d), lambda j: (0, 0)),
            pl.BlockSpec((1, d), lambda j: (0, 0)),
        ],
        out_specs=pl.BlockSpec((bj, d), lambda j: (j, 0)),
        out_shape=jax.ShapeDtypeStruct((n, d), jnp.float32),
        scratch_shapes=[pltpu.VMEM((d, n), jnp.float32)],
    )(cutoff, x, xT, ewT, Wl, bl.reshape(d, 1), Wr, br.reshape(1, d),
      We.reshape(1, d), att.reshape(1, d), bias.reshape(1, d))


def _conv_mat(w_ref, lin, lout, stride):
    im = jax.lax.broadcasted_iota(jnp.int32, (lin, lout), 0)
    il = jax.lax.broadcasted_iota(jnp.int32, (lin, lout), 1)
    a = jnp.zeros((lin, lout), jnp.float32)
    for k in range(5):
        a = a + jnp.where(im == stride * il + 3 * k, w_ref[0, k], jnp.float32(0.0))
    return a


def _tail_kernel(x_ref, w1_ref, b1_ref, w2_ref, b2_ref, w3_ref, b3_ref,
                 lw_ref, lb_ref, o_ref):
    xm = jnp.sum(x_ref[...], axis=0, keepdims=True) * jnp.float32(1.0 / _N)
    y = jnp.maximum(jnp.dot(xm, _conv_mat(w1_ref, 64, 52, 1),
                            preferred_element_type=jnp.float32) + b1_ref[0, 0], 0.0)
    y = jnp.maximum(jnp.dot(y, _conv_mat(w2_ref, 52, 40, 1),
                            preferred_element_type=jnp.float32) + b2_ref[0, 0], 0.0)
    y = jnp.maximum(jnp.dot(y, _conv_mat(w3_ref, 40, 14, 2),
                            preferred_element_type=jnp.float32) + b3_ref[0, 0], 0.0)
    o = jnp.sum(y * lw_ref[...], axis=1, keepdims=True) + lb_ref[0, 0]
    o_ref[...] = o


def _tail(x, c1_w, c1_b, c2_w, c2_b, c3_w, c3_b, l1_W, l1_b):
    smem = pl.BlockSpec(memory_space=pltpu.SMEM)
    vmem = pl.BlockSpec(memory_space=pltpu.VMEM)
    return pl.pallas_call(
        _tail_kernel,
        in_specs=[vmem, smem, smem, smem, smem, smem, smem, vmem, smem],
        out_specs=vmem,
        out_shape=jax.ShapeDtypeStruct((1, 1), jnp.float32),
    )(x, c1_w.reshape(1, 5), c1_b.reshape(1, 1), c2_w.reshape(1, 5),
      c2_b.reshape(1, 1), c3_w.reshape(1, 5), c3_b.reshape(1, 1),
      l1_W, l1_b.reshape(1, 1))


def kernel(features, edge_weights, threashold,
           conv1_Wl, conv1_bl, conv1_Wr, conv1_br, conv1_We, conv1_att, conv1_bias,
           conv2_Wl, conv2_bl, conv2_Wr, conv2_br, conv2_We, conv2_att, conv2_bias,
           conv3_Wl, conv3_bl, conv3_Wr, conv3_br, conv3_We, conv3_att, conv3_bias,
           conv4_Wl, conv4_bl, conv4_Wr, conv4_br, conv4_We, conv4_att, conv4_bias,
           c1_w, c1_b, c2_w, c2_b, c3_w, c3_b, l1_W, l1_b):
    cutoff = (jnp.float32(1.0) / threashold).astype(jnp.float32).reshape(1, 1)
    ewT = edge_weights.T
    x = features
    layer_ws = [
        (conv1_Wl, conv1_bl, conv1_Wr, conv1_br, conv1_We, conv1_att, conv1_bias),
        (conv2_Wl, conv2_bl, conv2_Wr, conv2_br, conv2_We, conv2_att, conv2_bias),
        (conv3_Wl, conv3_bl, conv3_Wr, conv3_br, conv3_We, conv3_att, conv3_bias),
        (conv4_Wl, conv4_bl, conv4_Wr, conv4_br, conv4_We, conv4_att, conv4_bias),
    ]
    for (Wl, bl, Wr, br, We, att, bias) in layer_ws:
        x = _gat_layer(x, x.T, ewT, cutoff, Wl, bl, Wr, br, We, att, bias)
    return _tail(x, c1_w, c1_b, c2_w, c2_b, c3_w, c3_b, l1_W, l1_b)


# speedup vs baseline: 275.2892x; 1.1401x over previous
import functools

import jax
import jax.numpy as jnp
from jax.experimental import pallas as pl
from jax.experimental.pallas import tpu as pltpu

_N = 1024


def _gat_layer_kernel(cut_ref, x_ref, xT_ref, ewT_ref, Wl_ref, bl_ref, Wr_ref,
                      br_ref, WeT_ref, att_ref, bias_ref, o_ref, xlT_s):
    cut = cut_ref[0, 0]
    ewT_b = ewT_ref[...]
    xT = xT_ref[...]
    x_b = x_ref[...]
    att = att_ref[...]
    WeT = WeT_ref[...]

    xlT = jnp.dot(Wl_ref[...], xT, preferred_element_type=jnp.float32) + bl_ref[...]
    xlT_s[...] = xlT
    xr_b = jax.lax.dot_general(x_b, Wr_ref[...], (((1,), (1,)), ((), ())),
                               preferred_element_type=jnp.float32) + br_ref[...]

    A_row = jnp.dot(att, xlT, preferred_element_type=jnp.float32)
    B_col = jax.lax.dot_general(xr_b, att, (((1,), (1,)), ((), ())),
                                preferred_element_type=jnp.float32)
    S = jax.lax.dot_general(att, WeT, (((1,), (1,)), ((), ())),
                            preferred_element_type=jnp.float32)
    lin = ewT_b * S + A_row + B_col

    d = att.shape[1]
    cchunk = 8

    def body(c0, acc):
        base = c0 * cchunk
        rows = xlT_s[pl.ds(base, cchunk), :]
        upd = None
        for k in range(cchunk):
            c = base + k
            sel = (jax.lax.broadcasted_iota(jnp.int32, (1, d), 1) == c).astype(jnp.float32)
            colc = jnp.sum(xr_b * sel, axis=1, keepdims=True)
            wec = jnp.sum(WeT * sel, axis=1, keepdims=True)
            attc = jnp.sum(att * sel, axis=1, keepdims=True)
            rowc = rows[k:k + 1, :]
            m = ewT_b * wec + rowc + colc
            t = jnp.abs(m) * attc
            upd = t if upd is None else upd + t
        return acc + upd

    T = jax.lax.fori_loop(0, d // cchunk, body, jnp.zeros_like(ewT_b))
    logits = 0.6 * lin + 0.4 * T

    neginf = jnp.float32(-jnp.inf)
    logits = jnp.where(ewT_b > cut, logits, neginf)
    rmax = jnp.max(logits, axis=1, keepdims=True)
    cm = jnp.where(rmax > neginf, rmax, jnp.float32(0.0))
    ex = jnp.exp(logits - cm)
    den = jnp.sum(ex, axis=1, keepdims=True)
    alpha = ex / (den + jnp.float32(1e-16))
    out = jax.lax.dot_general(alpha, xlT, (((1,), (1,)), ((), ())),
                              preferred_element_type=jnp.float32)
    o_ref[...] = jnp.maximum(out + bias_ref[...], jnp.float32(0.0))


def _gat_layer(x, xT, ewT, cutoff, Wl, bl, Wr, br, We, att, bias, *, bj=256):
    n, cin = x.shape
    d = Wl.shape[0]
    grid = (n // bj,)
    return pl.pallas_call(
        _gat_layer_kernel,
        grid=grid,
        in_specs=[
            pl.BlockSpec(memory_space=pltpu.SMEM),
            pl.BlockSpec((bj, cin), lambda j: (j, 0)),
            pl.BlockSpec((cin, n), lambda j: (0, 0)),
            pl.BlockSpec((bj, n), lambda j: (j, 0)),
            pl.BlockSpec((d, cin), lambda j: (0, 0)),
            pl.BlockSpec((d, 1), lambda j: (0, 0)),
            pl.BlockSpec((d, cin), lambda j: (0, 0)),
            pl.BlockSpec((1, d), lambda j: (0, 0)),
            pl.BlockSpec((1, d), lambda j: (0, 0)),
            pl.BlockSpec((1, d), lambda j: (0, 0)),
            pl.BlockSpec((1, d), lambda j: (0, 0)),
        ],
        out_specs=pl.BlockSpec((bj, d), lambda j: (j, 0)),
        out_shape=jax.ShapeDtypeStruct((n, d), jnp.float32),
        scratch_shapes=[pltpu.VMEM((d, n), jnp.float32)],
    )(cutoff, x, xT, ewT, Wl, bl.reshape(d, 1), Wr, br.reshape(1, d),
      We.reshape(1, d), att.reshape(1, d), bias.reshape(1, d))


def _conv_mat(w_ref, lin, lout, stride):
    im = jax.lax.broadcasted_iota(jnp.int32, (lin, lout), 0)
    il = jax.lax.broadcasted_iota(jnp.int32, (lin, lout), 1)
    a = jnp.zeros((lin, lout), jnp.float32)
    for k in range(5):
        a = a + jnp.where(im == stride * il + 3 * k, w_ref[0, k], jnp.float32(0.0))
    return a


def _tail_kernel(x_ref, w1_ref, b1_ref, w2_ref, b2_ref, w3_ref, b3_ref,
                 lw_ref, lb_ref, o_ref):
    xm = jnp.sum(x_ref[...], axis=0, keepdims=True) * jnp.float32(1.0 / _N)
    y = jnp.maximum(jnp.dot(xm, _conv_mat(w1_ref, 64, 52, 1),
                            preferred_element_type=jnp.float32) + b1_ref[0, 0], 0.0)
    y = jnp.maximum(jnp.dot(y, _conv_mat(w2_ref, 52, 40, 1),
                            preferred_element_type=jnp.float32) + b2_ref[0, 0], 0.0)
    y = jnp.maximum(jnp.dot(y, _conv_mat(w3_ref, 40, 14, 2),
                            preferred_element_type=jnp.float32) + b3_ref[0, 0], 0.0)
    o = jnp.sum(y * lw_ref[...], axis=1, keepdims=True) + lb_ref[0, 0]
    o_ref[...] = o


def _tail(x, c1_w, c1_b, c2_w, c2_b, c3_w, c3_b, l1_W, l1_b):
    smem = pl.BlockSpec(memory_space=pltpu.SMEM)
    vmem = pl.BlockSpec(memory_space=pltpu.VMEM)
    return pl.pallas_call(
        _tail_kernel,
        in_specs=[vmem, smem, smem, smem, smem, smem, smem, vmem, smem],
        out_specs=vmem,
        out_shape=jax.ShapeDtypeStruct((1, 1), jnp.float32),
    )(x, c1_w.reshape(1, 5), c1_b.reshape(1, 1), c2_w.reshape(1, 5),
      c2_b.reshape(1, 1), c3_w.reshape(1, 5), c3_b.reshape(1, 1),
      l1_W, l1_b.reshape(1, 1))


def kernel(features, edge_weights, threashold,
           conv1_Wl, conv1_bl, conv1_Wr, conv1_br, conv1_We, conv1_att, conv1_bias,
           conv2_Wl, conv2_bl, conv2_Wr, conv2_br, conv2_We, conv2_att, conv2_bias,
           conv3_Wl, conv3_bl, conv3_Wr, conv3_br, conv3_We, conv3_att, conv3_bias,
           conv4_Wl, conv4_bl, conv4_Wr, conv4_br, conv4_We, conv4_att, conv4_bias,
           c1_w, c1_b, c2_w, c2_b, c3_w, c3_b, l1_W, l1_b):
    cutoff = (jnp.float32(1.0) / threashold).astype(jnp.float32).reshape(1, 1)
    ewT = edge_weights.T
    x = features
    layer_ws = [
        (conv1_Wl, conv1_bl, conv1_Wr, conv1_br, conv1_We, conv1_att, conv1_bias),
        (conv2_Wl, conv2_bl, conv2_Wr, conv2_br, conv2_We, conv2_att, conv2_bias),
        (conv3_Wl, conv3_bl, conv3_Wr, conv3_br, conv3_We, conv3_att, conv3_bias),
        (conv4_Wl, conv4_bl, conv4_Wr, conv4_br, conv4_We, conv4_att, conv4_bias),
    ]
    for (Wl, bl, Wr, br, We, att, bias) in layer_ws:
        x = _gat_layer(x, x.T, ewT, cutoff, Wl, bl, Wr, br, We, att, bias)
    return _tail(x, c1_w, c1_b, c2_w, c2_b, c3_w, c3_b, l1_W, l1_b)
